# baseline (speedup 1.0000x reference)
import functools

import jax
import jax.numpy as jnp
from jax import lax
from jax.experimental import pallas as pl
from jax.experimental.pallas import tpu as pltpu
from jax.experimental.pallas import tpu_sc as plsc

_N_TOKENS = 32768
_E = 8
_L = 16
_NC, _NS = 2, 16
_NW = _NC * _NS
_TOTAL = _N_TOKENS * _E
_CHUNK = _TOTAL // _NW
_TOK_PER_W = _N_TOKENS // _NW
_STEPS = _TOK_PER_W // _L


def _routing_body(x_hbm, o_hbm, xv, ov):
    wid = lax.axis_index("s") * _NC + lax.axis_index("c")
    base = wid * _CHUNK
    pltpu.sync_copy(x_hbm.at[pl.ds(base, _CHUNK)], xv)

    iota = lax.iota(jnp.int32, _L)
    col_idx = iota * _E

    def step(i, carry):
        sbase = i * (_L * _E)
        idx0 = col_idx + sbase
        cols = [plsc.load_gather(xv, [idx0 + j]) for j in range(_E)]
        one = jnp.ones((_L,), jnp.int32)
        zero = jnp.zeros((_L,), jnp.int32)
        cnt = zero
        for j in range(1, _E):
            cnt = cnt + jnp.where(cols[j] > cols[0], one, zero)
        keep = cnt <= 1
        zf = jnp.zeros((_L,), jnp.float32)
        for j in range(_E):
            plsc.store_scatter(ov, [idx0 + j], jnp.where(keep, cols[j], zf))
        return carry

    lax.fori_loop(0, _STEPS, step, 0)
    pltpu.sync_copy(ov, o_hbm.at[pl.ds(base, _CHUNK)])


_routing = functools.partial(
    pl.kernel,
    mesh=plsc.VectorSubcoreMesh(core_axis_name="c", subcore_axis_name="s"),
    out_type=jax.ShapeDtypeStruct((_TOTAL,), jnp.float32),
    scratch_types=[
        pltpu.VMEM((_CHUNK,), jnp.float32),
        pltpu.VMEM((_CHUNK,), jnp.float32),
    ],
    compiler_params=pltpu.CompilerParams(
        use_tc_tiling_on_sc=False, needs_layout_passes=False
    ),
)(_routing_body)


@jax.jit
def kernel(x):
    out_flat = _routing(x.reshape(_TOTAL))
    return out_flat.reshape(_N_TOKENS, _E)
